# baseline (speedup 1.0000x reference)
import functools

import jax
import jax.numpy as jnp
from jax import lax
from jax.experimental import pallas as pl
from jax.experimental.pallas import tpu as pltpu
from jax.experimental.pallas import tpu_sc as plsc


_CHUNK = 512


@functools.lru_cache(maxsize=None)
def _build_gather(num_rows: int, emb: int, chunk: int):
    info = plsc.get_sparse_core_info()
    nw = info.num_cores * info.num_subcores
    assert num_rows % (nw * chunk) == 0
    rows_per_w = num_rows // nw
    n_chunks = rows_per_w // chunk
    mesh = plsc.VectorSubcoreMesh(core_axis_name="c", subcore_axis_name="s")

    @functools.partial(
        pl.kernel,
        mesh=mesh,
        out_type=jax.ShapeDtypeStruct((num_rows, emb), jnp.float32),
        scratch_types=[
            pltpu.VMEM((chunk,), jnp.int32),
            pltpu.VMEM((chunk, emb), jnp.float32),
            pltpu.SemaphoreType.DMA,
        ],
        compiler_params=pltpu.CompilerParams(use_tc_tiling_on_sc=False),
    )
    def gather(table_hbm, idx_hbm, out_hbm, idx_v, rows_v, sem):
        wid = lax.axis_index("s") * info.num_cores + lax.axis_index("c")
        wbase = wid * rows_per_w

        def body(c, carry):
            base = wbase + c * chunk
            pltpu.sync_copy(idx_hbm.at[pl.ds(base, chunk)], idx_v)
            pltpu.async_copy(table_hbm.at[idx_v], rows_v, sem).wait()
            pltpu.sync_copy(rows_v, out_hbm.at[pl.ds(base, chunk)])
            return carry

        lax.fori_loop(0, n_chunks, body, 0)

    return gather


def kernel(tokens, table):
    batch, max_len = tokens.shape
    emb = table.shape[1]
    flat = tokens.reshape(-1)
    out = _build_gather(flat.shape[0], emb, _CHUNK)(table, flat)
    return out.reshape(batch, max_len, emb)


# speedup vs baseline: 1.0435x; 1.0435x over previous
import functools

import jax
import jax.numpy as jnp
from jax import lax
from jax.experimental import pallas as pl
from jax.experimental.pallas import tpu as pltpu
from jax.experimental.pallas import tpu_sc as plsc


_CHUNK = 512
_NBUF = 2


@functools.lru_cache(maxsize=None)
def _build_gather(num_rows: int, emb: int, chunk: int, nbuf: int):
    info = plsc.get_sparse_core_info()
    nw = info.num_cores * info.num_subcores
    assert num_rows % (nw * chunk * nbuf) == 0
    rows_per_w = num_rows // nw
    n_chunks = rows_per_w // chunk
    n_groups = n_chunks // nbuf
    mesh = plsc.VectorSubcoreMesh(core_axis_name="c", subcore_axis_name="s")

    @functools.partial(
        pl.kernel,
        mesh=mesh,
        out_type=jax.ShapeDtypeStruct((num_rows, emb), jnp.float32),
        scratch_types=(
            [pltpu.VMEM((chunk,), jnp.int32) for _ in range(nbuf)]
            + [pltpu.VMEM((chunk, emb), jnp.float32) for _ in range(nbuf)]
            + [pltpu.SemaphoreType.DMA for _ in range(2 * nbuf)]
        ),
        compiler_params=pltpu.CompilerParams(use_tc_tiling_on_sc=False),
    )
    def gather(table_hbm, idx_hbm, out_hbm, *bufs):
        idx_v = bufs[:nbuf]
        rows_v = bufs[nbuf:2 * nbuf]
        sem_g = bufs[2 * nbuf:3 * nbuf]
        sem_s = bufs[3 * nbuf:4 * nbuf]
        wid = lax.axis_index("s") * info.num_cores + lax.axis_index("c")
        wbase = wid * rows_per_w

        for b in range(nbuf):
            pltpu.sync_copy(idx_hbm.at[pl.ds(wbase + b * chunk, chunk)],
                            idx_v[b])
            pltpu.async_copy(table_hbm.at[idx_v[b]], rows_v[b], sem_g[b])

        def group(g, carry):
            for b in range(nbuf):
                c = g * nbuf + b
                base = wbase + c * chunk
                pltpu.make_async_copy(table_hbm.at[idx_v[b]], rows_v[b],
                                      sem_g[b]).wait()
                pltpu.async_copy(rows_v[b], out_hbm.at[pl.ds(base, chunk)],
                                 sem_s[b])
                nc = c + nbuf

                @pl.when(nc < n_chunks)
                def _():
                    nbase = wbase + nc * chunk
                    pltpu.sync_copy(idx_hbm.at[pl.ds(nbase, chunk)], idx_v[b])
                    pltpu.make_async_copy(
                        rows_v[b], out_hbm.at[pl.ds(base, chunk)],
                        sem_s[b]).wait()
                    pltpu.async_copy(table_hbm.at[idx_v[b]], rows_v[b],
                                     sem_g[b])
            return carry

        lax.fori_loop(0, n_groups, group, 0)

        for b in range(nbuf):
            base = wbase + (n_chunks - nbuf + b) * chunk
            pltpu.make_async_copy(rows_v[b], out_hbm.at[pl.ds(base, chunk)],
                                  sem_s[b]).wait()

    return gather


def kernel(tokens, table):
    batch, max_len = tokens.shape
    emb = table.shape[1]
    flat = tokens.reshape(-1)
    out = _build_gather(flat.shape[0], emb, _CHUNK, _NBUF)(table, flat)
    return out.reshape(batch, max_len, emb)


# speedup vs baseline: 1.0439x; 1.0004x over previous
import functools

import jax
import jax.numpy as jnp
from jax import lax
from jax.experimental import pallas as pl
from jax.experimental.pallas import tpu as pltpu
from jax.experimental.pallas import tpu_sc as plsc


_CHUNK = 400
_NBUF = 4


@functools.lru_cache(maxsize=None)
def _build_gather(num_rows: int, emb: int, chunk: int, nbuf: int):
    info = plsc.get_sparse_core_info()
    nw = info.num_cores * info.num_subcores
    assert num_rows % (nw * chunk * nbuf) == 0
    rows_per_w = num_rows // nw
    n_chunks = rows_per_w // chunk
    n_groups = n_chunks // nbuf
    lag = nbuf - 1
    mesh = plsc.VectorSubcoreMesh(core_axis_name="c", subcore_axis_name="s")

    @functools.partial(
        pl.kernel,
        mesh=mesh,
        out_type=jax.ShapeDtypeStruct((num_rows, emb), jnp.float32),
        scratch_types=(
            [pltpu.VMEM((chunk,), jnp.int32) for _ in range(nbuf)]
            + [pltpu.VMEM((chunk, emb), jnp.float32) for _ in range(nbuf)]
            + [pltpu.SemaphoreType.DMA for _ in range(2 * nbuf)]
        ),
        compiler_params=pltpu.CompilerParams(use_tc_tiling_on_sc=False),
    )
    def gather(table_hbm, idx_hbm, out_hbm, *bufs):
        idx_v = bufs[:nbuf]
        rows_v = bufs[nbuf:2 * nbuf]
        sem_g = bufs[2 * nbuf:3 * nbuf]
        sem_s = bufs[3 * nbuf:4 * nbuf]
        wid = lax.axis_index("s") * info.num_cores + lax.axis_index("c")
        wbase = wid * rows_per_w

        def idx_fetch(c, b):
            pltpu.sync_copy(idx_hbm.at[pl.ds(wbase + c * chunk, chunk)],
                            idx_v[b])

        def gather_start(b):
            pltpu.async_copy(table_hbm.at[idx_v[b]], rows_v[b], sem_g[b])

        def gather_wait(b):
            pltpu.make_async_copy(table_hbm.at[idx_v[b]], rows_v[b],
                                  sem_g[b]).wait()

        def store_start(c, b):
            pltpu.async_copy(rows_v[b],
                             out_hbm.at[pl.ds(wbase + c * chunk, chunk)],
                             sem_s[b])

        def store_wait(c, b):
            pltpu.make_async_copy(rows_v[b],
                                  out_hbm.at[pl.ds(wbase + c * chunk, chunk)],
                                  sem_s[b]).wait()

        idx_fetch(0, 0)

        def group(g, carry):
            for b in range(nbuf):
                c = g * nbuf + b
                pl.when(c >= nbuf)(lambda: store_wait(c - nbuf, b))
                gather_start(b)
                d = (b + 1) % nbuf
                pl.when(c >= lag)(lambda: gather_wait(d))
                pl.when(c + 1 < n_chunks)(lambda: idx_fetch(c + 1, d))
                pl.when(c >= lag)(lambda: store_start(c - lag, d))
            return carry

        lax.fori_loop(0, n_groups, group, 0)

        for j in range(n_chunks - lag, n_chunks):
            b = j % nbuf
            gather_wait(b)
            store_start(j, b)
        for j in range(n_chunks - nbuf, n_chunks):
            store_wait(j, j % nbuf)

    return gather


def kernel(tokens, table):
    batch, max_len = tokens.shape
    emb = table.shape[1]
    flat = tokens.reshape(-1)
    out = _build_gather(flat.shape[0], emb, _CHUNK, _NBUF)(table, flat)
    return out.reshape(batch, max_len, emb)


# speedup vs baseline: 1.0447x; 1.0008x over previous
import functools

import jax
import jax.numpy as jnp
from jax import lax
from jax.experimental import pallas as pl
from jax.experimental.pallas import tpu as pltpu
from jax.experimental.pallas import tpu_sc as plsc


_RCHUNK = 2
_NBUF = 4


@functools.lru_cache(maxsize=None)
def _build_gather(batch: int, max_len: int, emb: int, rchunk: int, nbuf: int):
    info = plsc.get_sparse_core_info()
    nw = info.num_cores * info.num_subcores
    assert batch % (nw * rchunk * nbuf) == 0
    rows_per_w = batch // nw
    n_chunks = rows_per_w // rchunk
    n_groups = n_chunks // nbuf
    lag = nbuf - 1
    mesh = plsc.VectorSubcoreMesh(core_axis_name="c", subcore_axis_name="s")

    @functools.partial(
        pl.kernel,
        mesh=mesh,
        out_type=jax.ShapeDtypeStruct((batch, max_len, emb), jnp.float32),
        scratch_types=(
            [pltpu.VMEM((rchunk * max_len,), jnp.int32) for _ in range(nbuf)]
            + [pltpu.VMEM((rchunk * max_len, emb), jnp.float32)
               for _ in range(nbuf)]
            + [pltpu.SemaphoreType.DMA for _ in range(2 * nbuf)]
        ),
        compiler_params=pltpu.CompilerParams(use_tc_tiling_on_sc=False),
    )
    def gather(table_hbm, idx_hbm, out_hbm, *bufs):
        idx_v = bufs[:nbuf]
        rows_v = bufs[nbuf:2 * nbuf]
        sem_g = bufs[2 * nbuf:3 * nbuf]
        sem_s = bufs[3 * nbuf:4 * nbuf]
        wid = lax.axis_index("s") * info.num_cores + lax.axis_index("c")
        wbase = wid * rows_per_w

        n_idx = rchunk * max_len

        def idx_fetch(c, b):
            pltpu.sync_copy(
                idx_hbm.at[pl.ds((wbase + c * rchunk) * max_len, n_idx)],
                idx_v[b])

        def gather_start(b):
            pltpu.async_copy(table_hbm.at[idx_v[b]], rows_v[b], sem_g[b])

        def gather_wait(b):
            pltpu.make_async_copy(table_hbm.at[idx_v[b]], rows_v[b],
                                  sem_g[b]).wait()

        def store_start(c, b):
            for r in range(rchunk):
                pltpu.async_copy(rows_v[b].at[pl.ds(r * max_len, max_len)],
                                 out_hbm.at[wbase + c * rchunk + r],
                                 sem_s[b])

        def store_wait(c, b):
            for r in range(rchunk):
                pltpu.make_async_copy(
                    rows_v[b].at[pl.ds(r * max_len, max_len)],
                    out_hbm.at[wbase + c * rchunk + r],
                    sem_s[b]).wait()

        idx_fetch(0, 0)

        def group(g, carry):
            for b in range(nbuf):
                c = g * nbuf + b
                pl.when(c >= nbuf)(lambda: store_wait(c - nbuf, b))
                gather_start(b)
                d = (b + 1) % nbuf
                pl.when(c >= lag)(lambda: gather_wait(d))
                pl.when(c + 1 < n_chunks)(lambda: idx_fetch(c + 1, d))
                pl.when(c >= lag)(lambda: store_start(c - lag, d))
            return carry

        lax.fori_loop(0, n_groups, group, 0)

        for j in range(n_chunks - lag, n_chunks):
            b = j % nbuf
            gather_wait(b)
            store_start(j, b)
        for j in range(n_chunks - nbuf, n_chunks):
            store_wait(j, j % nbuf)

    return gather


def kernel(tokens, table):
    batch, max_len = tokens.shape
    emb = table.shape[1]
    flat = tokens.reshape(-1)
    return _build_gather(batch, max_len, emb, _RCHUNK, _NBUF)(table, flat)


# speedup vs baseline: 1.0458x; 1.0010x over previous
import functools

import jax
import jax.numpy as jnp
from jax import lax
from jax.experimental import pallas as pl
from jax.experimental.pallas import tpu as pltpu
from jax.experimental.pallas import tpu_sc as plsc


_RCHUNK = 2
_NBUF = 4


@functools.lru_cache(maxsize=None)
def _build_gather(batch: int, max_len: int, emb: int, rchunk: int, nbuf: int):
    info = plsc.get_sparse_core_info()
    nw = info.num_cores * info.num_subcores
    assert batch % (nw * rchunk * nbuf) == 0
    rows_per_w = batch // nw
    n_chunks = rows_per_w // rchunk
    n_groups = n_chunks // nbuf
    lag = nbuf - 1
    mesh = plsc.VectorSubcoreMesh(core_axis_name="c", subcore_axis_name="s")

    @functools.partial(
        pl.kernel,
        mesh=mesh,
        out_type=jax.ShapeDtypeStruct((batch, max_len, emb), jnp.float32),
        scratch_types=(
            [pltpu.VMEM((rchunk, max_len), jnp.int32) for _ in range(nbuf)]
            + [pltpu.VMEM((rchunk, max_len, emb), jnp.float32)
               for _ in range(nbuf)]
            + [pltpu.SemaphoreType.DMA for _ in range(2 * nbuf)]
        ),
        compiler_params=pltpu.CompilerParams(use_tc_tiling_on_sc=False),
    )
    def gather(table_hbm, idx_hbm, out_hbm, *bufs):
        idx_v = bufs[:nbuf]
        rows_v = bufs[nbuf:2 * nbuf]
        sem_g = bufs[2 * nbuf:3 * nbuf]
        sem_s = bufs[3 * nbuf:4 * nbuf]
        wid = lax.axis_index("s") * info.num_cores + lax.axis_index("c")
        wbase = wid * rows_per_w

        def idx_fetch(c, b):
            pltpu.sync_copy(idx_hbm.at[pl.ds(wbase + c * rchunk, rchunk)],
                            idx_v[b])

        def gather_start(b):
            for r in range(rchunk):
                pltpu.async_copy(table_hbm.at[idx_v[b].at[r]],
                                 rows_v[b].at[r], sem_g[b])

        def gather_wait(b):
            for r in range(rchunk):
                pltpu.make_async_copy(table_hbm.at[idx_v[b].at[r]],
                                      rows_v[b].at[r], sem_g[b]).wait()

        def store_start(c, b):
            pltpu.async_copy(rows_v[b],
                             out_hbm.at[pl.ds(wbase + c * rchunk, rchunk)],
                             sem_s[b])

        def store_wait(c, b):
            pltpu.make_async_copy(rows_v[b],
                                  out_hbm.at[pl.ds(wbase + c * rchunk, rchunk)],
                                  sem_s[b]).wait()

        idx_fetch(0, 0)

        def group(g, carry):
            for b in range(nbuf):
                c = g * nbuf + b
                pl.when(c >= nbuf)(lambda: store_wait(c - nbuf, b))
                gather_start(b)
                d = (b + 1) % nbuf
                pl.when(c >= lag)(lambda: gather_wait(d))
                pl.when(c + 1 < n_chunks)(lambda: idx_fetch(c + 1, d))
                pl.when(c >= lag)(lambda: store_start(c - lag, d))
            return carry

        lax.fori_loop(0, n_groups, group, 0)

        for j in range(n_chunks - lag, n_chunks):
            b = j % nbuf
            gather_wait(b)
            store_start(j, b)
        for j in range(n_chunks - nbuf, n_chunks):
            store_wait(j, j % nbuf)

    return gather


def kernel(tokens, table):
    batch, max_len = tokens.shape
    emb = table.shape[1]
    return _build_gather(batch, max_len, emb, _RCHUNK, _NBUF)(table, tokens)


# speedup vs baseline: 1.3920x; 1.3311x over previous
import functools

import jax
import jax.numpy as jnp
from jax import lax
from jax.experimental import pallas as pl
from jax.experimental.pallas import tpu as pltpu
from jax.experimental.pallas import tpu_sc as plsc


_RCHUNK = 2
_NBUF = 4


@functools.lru_cache(maxsize=None)
def _build_gather(batch: int, max_len: int, emb: int, rchunk: int, nbuf: int):
    info = plsc.get_sparse_core_info()
    nw = info.num_cores * info.num_subcores
    assert batch % (nw * rchunk * nbuf) == 0
    rows_per_w = batch // nw
    n_chunks = rows_per_w // rchunk
    n_groups = n_chunks // nbuf
    lag = nbuf - 1
    mesh = plsc.VectorSubcoreMesh(core_axis_name="c", subcore_axis_name="s")

    @functools.partial(
        pl.kernel,
        mesh=mesh,
        out_type=jax.ShapeDtypeStruct((batch, max_len, 128), jnp.float32),
        scratch_types=(
            [pltpu.VMEM((rchunk, max_len), jnp.int32) for _ in range(nbuf)]
            + [pltpu.VMEM((rchunk, max_len, emb), jnp.float32)
               for _ in range(nbuf)]
            + [pltpu.SemaphoreType.DMA for _ in range(2 * nbuf)]
        ),
        compiler_params=pltpu.CompilerParams(use_tc_tiling_on_sc=False),
    )
    def gather(table_hbm, idx_hbm, out_hbm, *bufs):
        idx_v = bufs[:nbuf]
        rows_v = bufs[nbuf:2 * nbuf]
        sem_g = bufs[2 * nbuf:3 * nbuf]
        sem_s = bufs[3 * nbuf:4 * nbuf]
        wid = lax.axis_index("s") * info.num_cores + lax.axis_index("c")
        wbase = wid * rows_per_w

        def idx_fetch(c, b):
            pltpu.sync_copy(idx_hbm.at[pl.ds(wbase + c * rchunk, rchunk)],
                            idx_v[b])

        def gather_start(b):
            for r in range(rchunk):
                pltpu.async_copy(table_hbm.at[idx_v[b].at[r]],
                                 rows_v[b].at[r], sem_g[b])

        def gather_wait(b):
            for r in range(rchunk):
                pltpu.make_async_copy(table_hbm.at[idx_v[b].at[r]],
                                      rows_v[b].at[r], sem_g[b]).wait()

        def store_start(c, b):
            pltpu.async_copy(
                rows_v[b],
                out_hbm.at[pl.ds(wbase + c * rchunk, rchunk), :, pl.ds(0, emb)],
                sem_s[b])

        def store_wait(c, b):
            pltpu.make_async_copy(
                rows_v[b],
                out_hbm.at[pl.ds(wbase + c * rchunk, rchunk), :, pl.ds(0, emb)],
                sem_s[b]).wait()

        idx_fetch(0, 0)

        def group(g, carry):
            for b in range(nbuf):
                c = g * nbuf + b
                pl.when(c >= nbuf)(lambda: store_wait(c - nbuf, b))
                gather_start(b)
                d = (b + 1) % nbuf
                pl.when(c >= lag)(lambda: gather_wait(d))
                pl.when(c + 1 < n_chunks)(lambda: idx_fetch(c + 1, d))
                pl.when(c >= lag)(lambda: store_start(c - lag, d))
            return carry

        lax.fori_loop(0, n_groups, group, 0)

        for j in range(n_chunks - lag, n_chunks):
            b = j % nbuf
            gather_wait(b)
            store_start(j, b)
        for j in range(n_chunks - nbuf, n_chunks):
            store_wait(j, j % nbuf)

    return gather


def kernel(tokens, table):
    batch, max_len = tokens.shape
    emb = table.shape[1]
    out_p = _build_gather(batch, max_len, emb, _RCHUNK, _NBUF)(table, tokens)
    return out_p[:, :, :emb]
